# baseline (speedup 1.0000x reference)
import jax
import jax.numpy as jnp
import numpy as np
from jax.experimental import pallas as pl
from jax.experimental.pallas import tpu as pltpu

_ANCHORS = np.array(
    [[12, 16], [19, 36], [40, 28], [36, 75], [76, 55], [72, 146],
     [142, 110], [192, 243], [459, 401]], dtype=np.float32)
_ANCHOR_MASKS = [[6, 7, 8], [3, 4, 5], [0, 1, 2]]
_DOWNSAMPLE = [32.0, 16.0, 8.0]
_OC = 85
_NC = 255


def _emit(x_ref, o_ref, a, s, nx, ds, anc):
    y = x_ref[...]
    ci = jax.lax.broadcasted_iota(jnp.int32, (_OC, _NC), 0)
    ki = jax.lax.broadcasted_iota(jnp.int32, (_OC, _NC), 1)
    e = (ki == ci + a * _OC).astype(jnp.float32)
    raw = jax.lax.dot_general(e, y, (((1,), (2,)), ((), ())),
                              precision=jax.lax.Precision.DEFAULT)
    sig = 0.5 * jnp.tanh(0.5 * raw) + 0.5
    head = raw[0:8]
    c = jax.lax.broadcasted_iota(jnp.int32, head.shape, 0)
    p = s * 256 + jax.lax.broadcasted_iota(jnp.int32, head.shape, 2)
    gx = (p % nx).astype(jnp.float32)
    gy = (p // nx).astype(jnp.float32)
    g = jnp.where(c == 0, gx, gy)
    aw = jnp.where(a == 0, anc[0][0], jnp.where(a == 1, anc[1][0], anc[2][0]))
    ah = jnp.where(a == 0, anc[0][1], jnp.where(a == 1, anc[1][1], anc[2][1]))
    av = jnp.where(c == 2, aw, ah)
    xywh = jnp.where(c < 2, (sig[0:8] + g) * ds, jnp.exp(head) * av)
    o_ref[0:8] = jnp.where(c < 4, xywh, sig[0:8])
    o_ref[8:_OC] = sig[8:_OC]


def _body(x0_ref, x1_ref, x2_ref, o_ref):
    j = pl.program_id(0)

    @pl.when(j < 3)
    def _():
        _emit(x0_ref, o_ref, j, 0, 16, _DOWNSAMPLE[0],
              _ANCHORS[np.array(_ANCHOR_MASKS[0])])

    @pl.when((j >= 3) & (j < 15))
    def _():
        t = j - 3
        _emit(x1_ref, o_ref, jax.lax.rem(t, 3), t // 3, 32, _DOWNSAMPLE[1],
              _ANCHORS[np.array(_ANCHOR_MASKS[1])])

    @pl.when(j >= 15)
    def _():
        t = j - 15
        _emit(x2_ref, o_ref, jax.lax.rem(t, 3), t // 3, 64, _DOWNSAMPLE[2],
              _ANCHORS[np.array(_ANCHOR_MASKS[2])])


def _out_chunk(j):
    t1 = jnp.clip(j - 3, 0, 11)
    t2 = jnp.clip(j - 15, 0, 47)
    return jnp.where(
        j < 3, j,
        jnp.where(j < 15,
                  3 + (t1 % 3) * 4 + t1 // 3,
                  15 + (t2 % 3) * 16 + t2 // 3))


def kernel(x0, x1, x2):
    bs = x0.shape[0]
    xt = [
        jnp.transpose(x, (0, 2, 3, 1)).reshape(
            bs, x.shape[-2] * x.shape[-1], _NC)
        for x in (x0, x1, x2)
    ]
    o = pl.pallas_call(
        _body,
        grid=(63,),
        in_specs=[
            pl.BlockSpec((bs, 256, _NC), lambda j: (0, 0, 0)),
            pl.BlockSpec((bs, 256, _NC),
                         lambda j: (0, jnp.clip(j - 3, 0, 11) // 3, 0)),
            pl.BlockSpec((bs, 256, _NC),
                         lambda j: (0, jnp.clip(j - 15, 0, 47) // 3, 0)),
        ],
        out_specs=pl.BlockSpec((_OC, bs, 256), lambda j: (0, 0, _out_chunk(j))),
        out_shape=jax.ShapeDtypeStruct((_OC, bs, 63 * 256), jnp.float32),
        compiler_params=pltpu.CompilerParams(
            dimension_semantics=("parallel",)),
    )(*xt)
    return jnp.transpose(o, (1, 2, 0))


# speedup vs baseline: 1.0308x; 1.0308x over previous
import jax
import jax.numpy as jnp
import numpy as np
from jax.experimental import pallas as pl

_ANCHORS = np.array(
    [[12, 16], [19, 36], [40, 28], [36, 75], [76, 55], [72, 146],
     [142, 110], [192, 243], [459, 401]], dtype=np.float32)
_ANCHOR_MASKS = [[6, 7, 8], [3, 4, 5], [0, 1, 2]]
_DOWNSAMPLE = [32.0, 16.0, 8.0]
_OC = 85
_NC = 255


def _emit(x_ref, o_ref, a, s, nx, ds, anc):
    if a == 1:
        y = x_ref[...]
        k_dim, coff = _NC, _OC
    else:
        k0 = 0 if a == 0 else 128
        k_dim = 128 if a == 0 else _NC - 128
        y = x_ref[:, :, k0:k0 + k_dim]
        coff = a * _OC - k0
    ci = jax.lax.broadcasted_iota(jnp.int32, (_OC, k_dim), 0)
    ki = jax.lax.broadcasted_iota(jnp.int32, (_OC, k_dim), 1)
    e = (ki == ci + coff).astype(jnp.float32)
    raw = jax.lax.dot_general(e, y, (((1,), (2,)), ((), ())),
                              precision=jax.lax.Precision.DEFAULT)
    sig = 0.5 * jnp.tanh(0.5 * raw) + 0.5
    head = raw[0:8]
    c = jax.lax.broadcasted_iota(jnp.int32, head.shape, 0)
    p = s * 256 + jax.lax.broadcasted_iota(jnp.int32, head.shape, 2)
    gx = (p % nx).astype(jnp.float32)
    gy = (p // nx).astype(jnp.float32)
    g = jnp.where(c == 0, gx, gy)
    av = jnp.where(c == 2, float(anc[a][0]), float(anc[a][1]))
    xywh = jnp.where(c < 2, (sig[0:8] + g) * ds, jnp.exp(head) * av)
    o_ref[0:8] = jnp.where(c < 4, xywh, sig[0:8])
    o_ref[8:_OC] = sig[8:_OC]


def _body(x0_ref, x1_ref, x2_ref, o_ref):
    j = pl.program_id(0)

    for a in range(3):
        @pl.when(j == a)
        def _(a=a):
            _emit(x0_ref, o_ref, a, 0, 16, _DOWNSAMPLE[0],
                  _ANCHORS[np.array(_ANCHOR_MASKS[0])])

        @pl.when((j >= 3) & (j < 15) & (jax.lax.rem(j - 3, 3) == a))
        def _(a=a):
            _emit(x1_ref, o_ref, a, (j - 3) // 3, 32, _DOWNSAMPLE[1],
                  _ANCHORS[np.array(_ANCHOR_MASKS[1])])

        @pl.when((j >= 15) & (jax.lax.rem(j - 15, 3) == a))
        def _(a=a):
            _emit(x2_ref, o_ref, a, (j - 15) // 3, 64, _DOWNSAMPLE[2],
                  _ANCHORS[np.array(_ANCHOR_MASKS[2])])


def _out_chunk(j):
    t1 = jnp.clip(j - 3, 0, 11)
    t2 = jnp.clip(j - 15, 0, 47)
    return jnp.where(
        j < 3, j,
        jnp.where(j < 15,
                  3 + (t1 % 3) * 4 + t1 // 3,
                  15 + (t2 % 3) * 16 + t2 // 3))


def kernel(x0, x1, x2):
    bs = x0.shape[0]
    xt = [
        jnp.transpose(x, (0, 2, 3, 1)).reshape(
            bs, x.shape[-2] * x.shape[-1], _NC)
        for x in (x0, x1, x2)
    ]
    o = pl.pallas_call(
        _body,
        grid=(63,),
        in_specs=[
            pl.BlockSpec((bs, 256, _NC), lambda j: (0, 0, 0)),
            pl.BlockSpec((bs, 256, _NC),
                         lambda j: (0, jnp.clip(j - 3, 0, 11) // 3, 0)),
            pl.BlockSpec((bs, 256, _NC),
                         lambda j: (0, jnp.clip(j - 15, 0, 47) // 3, 0)),
        ],
        out_specs=pl.BlockSpec((_OC, bs, 256), lambda j: (0, 0, _out_chunk(j))),
        out_shape=jax.ShapeDtypeStruct((_OC, bs, 63 * 256), jnp.float32),
    )(*xt)
    return jnp.transpose(o, (1, 2, 0))
